# baseline (speedup 1.0000x reference)
import functools
import math

import jax
import jax.numpy as jnp
from jax import lax
from jax.experimental import pallas as pl
from jax.experimental.pallas import tpu as pltpu
from jax.experimental.pallas import tpu_sc as plsc

B, C, H, W = 16, 100, 32, 32
N = H * W
K = 9
HID = 2 * C
FFN_HID = 4 * C
EPS = 1e-5

CP = 128
R = B * N
_F32 = jnp.float32
_HIGH = lax.Precision.HIGHEST


def _dot(a, b, dims, precision=_HIGH):
    return lax.dot_general(a, b, (dims, ((), ())), precision=precision,
                           preferred_element_type=_F32)


def _gelu(u):
    return 0.5 * u * (1.0 + lax.erf(u * (1.0 / math.sqrt(2.0))))


def _stage1_body(x_ref, w1_ref, b1_ref, g1_ref, be1_ref, ht_ref, idx_ref):
    xb = x_ref[0]
    conv = _dot(xb, w1_ref[...], (((1,), (0,))), precision=lax.Precision.DEFAULT)
    ht = (conv + b1_ref[...]) / jnp.sqrt(_F32(1.0) + _F32(EPS)) * g1_ref[...] \
        + be1_ref[...]

    x2 = jnp.sum(ht * ht, axis=1)
    inner = _dot(ht, ht, (((1,), (1,))), precision=lax.Precision.DEFAULT)
    dist = x2[:, None] - 2.0 * inner + x2[None, :]

    cols = lax.broadcasted_iota(jnp.int32, (N, N), 1)
    picks = []
    for _ in range(K):
        m = jnp.min(dist, axis=1, keepdims=True)
        eq = dist == m
        jsel = jnp.min(jnp.where(eq, cols, N), axis=1, keepdims=True)
        picks.append(jsel)
        dist = jnp.where(cols == jsel, jnp.inf, dist)

    base = pl.program_id(0) * N
    idx_ref[0] = jnp.concatenate(picks, axis=1) + base
    ht_ref[0] = jnp.concatenate(
        [ht, jnp.zeros((N, CP - C), _F32)], axis=1)


def _make_sc_gather():
    info = plsc.get_sparse_core_info()
    nw = info.num_cores * info.num_subcores
    rows_per_w = R // nw
    G = 8
    chunks = rows_per_w // G

    mesh = plsc.VectorSubcoreMesh(core_axis_name="c", subcore_axis_name="s")

    @functools.partial(
        pl.kernel, mesh=mesh,
        out_type=jax.ShapeDtypeStruct((R, CP), jnp.float32),
        scratch_types=[
            pltpu.VMEM((rows_per_w * K,), jnp.int32),
            pltpu.VMEM((G * K, CP), jnp.float32),
            pltpu.VMEM((G * K, CP), jnp.float32),
            pltpu.VMEM((G * K, CP), jnp.float32),
            pltpu.VMEM((G * K, CP), jnp.float32),
            pltpu.VMEM((rows_per_w, CP), jnp.float32),
            pltpu.SemaphoreType.DMA,
            pltpu.SemaphoreType.DMA,
            pltpu.SemaphoreType.DMA,
            pltpu.SemaphoreType.DMA,
        ],
    )
    def sc_gather(table_hbm, idx_hbm, out_hbm, idx_v,
                  rows_a, rows_b, rows_c, rows_d, out_v,
                  sem_a, sem_b, sem_c, sem_d):
        wid = lax.axis_index("s") * info.num_cores + lax.axis_index("c")
        base = wid * rows_per_w
        pltpu.sync_copy(idx_hbm.at[pl.ds(base * K, rows_per_w * K)], idx_v)

        def reduce_chunk(ci, rows_v):
            for i in range(G):
                for cc in range(CP // 16):
                    sl = pl.ds(cc * 16, 16)
                    acc = rows_v[K * i, sl]
                    for kk in range(1, K):
                        acc = jnp.maximum(acc, rows_v[K * i + kk, sl])
                    out_v[ci * G + i, sl] = acc

        bufs = ((rows_a, sem_a), (rows_b, sem_b), (rows_c, sem_c),
                (rows_d, sem_d))

        def quad(cq, carry):
            ci = cq * 4
            copies = [
                pltpu.async_copy(
                    table_hbm.at[idx_v.at[pl.ds((ci + j) * G * K, G * K)]],
                    buf, sem)
                for j, (buf, sem) in enumerate(bufs)
            ]
            for j, (buf, _) in enumerate(bufs):
                copies[j].wait()
                reduce_chunk(ci + j, buf)
            return carry

        lax.fori_loop(0, chunks // 4, quad, 0)
        pltpu.sync_copy(out_v, out_hbm.at[pl.ds(base, rows_per_w)])

    return sc_gather


_sc_gather = _make_sc_gather()


def _stage3_body(x_ref, ht_ref, rm_ref, wa_ref, wm_ref, bgc_ref, w2_ref, b2_ref,
                 f1_ref, bf1_ref, f2_ref, bf2_ref, out_ref):
    xb = x_ref[0]
    ht = ht_ref[0][:, :C]
    relmax = rm_ref[0][:, :C]

    u = (_dot(ht, wa_ref[...], (((1,), (0,))))
         + _dot(relmax, wm_ref[...], (((1,), (0,))))
         + bgc_ref[...])
    u = _gelu(u)
    h4 = _dot(u, w2_ref[...], (((1,), (0,)))) + b2_ref[...]
    y1 = h4 + xb

    v = _gelu(_dot(y1, f1_ref[...], (((1,), (0,)))) + bf1_ref[...])
    y2 = _dot(v, f2_ref[...], (((1,), (0,)))) + bf2_ref[...] + y1
    out_ref[0] = y2


def kernel(x, g_fc1_w, g_fc1_b, g_bn1_g, g_bn1_b, gc_w, gc_b, gc_bn_g, gc_bn_b,
           g_fc2_w, g_fc2_b, g_bn2_g, g_bn2_b,
           f_fc1_w, f_fc1_b, f_bn1_g, f_bn1_b, f_fc2_w, f_fc2_b, f_bn2_g, f_bn2_b):
    s = 1.0 / jnp.sqrt(jnp.float32(1.0 + EPS))

    def fold(w, b, g, be):
        sc = s * g
        return (w * sc[:, None]).T, (b * sc + be)[None, :]

    gcw, bgc = fold(gc_w, gc_b, gc_bn_g, gc_bn_b)
    wa = gcw[:C] - gcw[C:]
    wm = gcw[C:]
    w2, b2 = fold(g_fc2_w, g_fc2_b, g_bn2_g, g_bn2_b)
    f1, bf1 = fold(f_fc1_w, f_fc1_b, f_bn1_g, f_bn1_b)
    f2, bf2 = fold(f_fc2_w, f_fc2_b, f_bn2_g, f_bn2_b)

    xt = x.reshape(B, C, N).transpose(0, 2, 1)

    def fixed(shape):
        return pl.BlockSpec(shape, lambda b: (0,) * len(shape))

    ht_pad, idx = pl.pallas_call(
        _stage1_body,
        grid=(B,),
        in_specs=[
            pl.BlockSpec((1, N, C), lambda b: (b, 0, 0)),
            fixed((C, C)), fixed((1, C)), fixed((1, C)), fixed((1, C)),
        ],
        out_specs=[
            pl.BlockSpec((1, N, CP), lambda b: (b, 0, 0)),
            pl.BlockSpec((1, N, K), lambda b: (b, 0, 0)),
        ],
        out_shape=[
            jax.ShapeDtypeStruct((B, N, CP), _F32),
            jax.ShapeDtypeStruct((B, N, K), jnp.int32),
        ],
    )(xt, g_fc1_w.T, g_fc1_b[None, :], g_bn1_g[None, :], g_bn1_b[None, :])

    relmax = _sc_gather(ht_pad.reshape(R, CP), idx.reshape(R * K))

    out = pl.pallas_call(
        _stage3_body,
        grid=(B,),
        in_specs=[
            pl.BlockSpec((1, N, C), lambda b: (b, 0, 0)),
            pl.BlockSpec((1, N, CP), lambda b: (b, 0, 0)),
            pl.BlockSpec((1, N, CP), lambda b: (b, 0, 0)),
            fixed((C, HID)), fixed((C, HID)), fixed((1, HID)),
            fixed((HID, C)), fixed((1, C)),
            fixed((C, FFN_HID)), fixed((1, FFN_HID)),
            fixed((FFN_HID, C)), fixed((1, C)),
        ],
        out_specs=pl.BlockSpec((1, N, C), lambda b: (b, 0, 0)),
        out_shape=jax.ShapeDtypeStruct((B, N, C), _F32),
    )(xt, ht_pad, relmax.reshape(B, N, CP), wa, wm, bgc, w2, b2, f1, bf1, f2, bf2)

    return out.transpose(0, 2, 1).reshape(B, C, H, W)


# speedup vs baseline: 1.0306x; 1.0306x over previous
import functools
import math

import jax
import jax.numpy as jnp
from jax import lax
from jax.experimental import pallas as pl
from jax.experimental.pallas import tpu as pltpu
from jax.experimental.pallas import tpu_sc as plsc

B, C, H, W = 16, 100, 32, 32
N = H * W
K = 9
HID = 2 * C
FFN_HID = 4 * C
EPS = 1e-5

CP = 128
R = B * N
_F32 = jnp.float32
_HIGH = lax.Precision.HIGHEST


def _dot(a, b, dims, precision=_HIGH):
    return lax.dot_general(a, b, (dims, ((), ())), precision=precision,
                           preferred_element_type=_F32)


def _gelu(u):
    return 0.5 * u * (1.0 + lax.erf(u * (1.0 / math.sqrt(2.0))))


def _stage1_body(x_ref, w1_ref, b1_ref, g1_ref, be1_ref, ht_ref, idx_ref):
    xb = x_ref[0]
    conv = _dot(xb, w1_ref[...], (((1,), (0,))), precision=lax.Precision.DEFAULT)
    ht = (conv + b1_ref[...]) / jnp.sqrt(_F32(1.0) + _F32(EPS)) * g1_ref[...] \
        + be1_ref[...]

    x2 = jnp.sum(ht * ht, axis=1)
    inner = _dot(ht, ht, (((1,), (1,))), precision=lax.Precision.DEFAULT)
    dist = x2[:, None] - 2.0 * inner + x2[None, :]

    cols = lax.broadcasted_iota(jnp.int32, (N, N), 1)
    picks = []
    for _ in range(K):
        m = jnp.min(dist, axis=1, keepdims=True)
        eq = dist == m
        jsel = jnp.min(jnp.where(eq, cols, N), axis=1, keepdims=True)
        picks.append(jsel)
        dist = jnp.where(cols == jsel, jnp.inf, dist)

    base = pl.program_id(0) * N
    idx_ref[0] = jnp.concatenate(picks, axis=1) + base
    ht_ref[0] = jnp.concatenate(
        [ht, jnp.zeros((N, CP - C), _F32)], axis=1)


def _make_sc_gather():
    info = plsc.get_sparse_core_info()
    nw = info.num_cores * info.num_subcores
    rows_per_w = R // nw
    G = 8
    chunks = rows_per_w // G

    mesh = plsc.VectorSubcoreMesh(core_axis_name="c", subcore_axis_name="s")

    @functools.partial(
        pl.kernel, mesh=mesh,
        out_type=jax.ShapeDtypeStruct((R, CP), jnp.float32),
        scratch_types=[
            pltpu.VMEM((rows_per_w * K,), jnp.int32),
            pltpu.VMEM((G * K, CP), jnp.float32),
            pltpu.VMEM((G * K, CP), jnp.float32),
            pltpu.VMEM((rows_per_w, CP), jnp.float32),
            pltpu.SemaphoreType.DMA,
            pltpu.SemaphoreType.DMA,
        ],
    )
    def sc_gather(table_hbm, idx_hbm, out_hbm, idx_v, rows_a, rows_b, out_v,
                  sem_a, sem_b):
        wid = lax.axis_index("s") * info.num_cores + lax.axis_index("c")
        base = wid * rows_per_w
        pltpu.sync_copy(idx_hbm.at[pl.ds(base * K, rows_per_w * K)], idx_v)

        def reduce_chunk(ci, rows_v):
            for i in range(G):
                for cc in range(CP // 16):
                    sl = pl.ds(cc * 16, 16)
                    acc = rows_v[K * i, sl]
                    for kk in range(1, K):
                        acc = jnp.maximum(acc, rows_v[K * i + kk, sl])
                    out_v[ci * G + i, sl] = acc

        def pair(cp, carry):
            ci = cp * 2
            cp_a = pltpu.async_copy(
                table_hbm.at[idx_v.at[pl.ds(ci * G * K, G * K)]], rows_a, sem_a)
            cp_b = pltpu.async_copy(
                table_hbm.at[idx_v.at[pl.ds((ci + 1) * G * K, G * K)]],
                rows_b, sem_b)
            cp_a.wait()
            reduce_chunk(ci, rows_a)
            cp_b.wait()
            reduce_chunk(ci + 1, rows_b)
            return carry

        lax.fori_loop(0, chunks // 2, pair, 0)
        pltpu.sync_copy(out_v, out_hbm.at[pl.ds(base, rows_per_w)])

    return sc_gather


_sc_gather = _make_sc_gather()


def _stage3_body(x_ref, ht_ref, rm_ref, wa_ref, wm_ref, bgc_ref, w2_ref, b2_ref,
                 f1_ref, bf1_ref, f2_ref, bf2_ref, out_ref):
    xb = x_ref[0]
    ht = ht_ref[0][:, :C]
    relmax = rm_ref[0][:, :C]

    u = (_dot(ht, wa_ref[...], (((1,), (0,))))
         + _dot(relmax, wm_ref[...], (((1,), (0,))))
         + bgc_ref[...])
    u = _gelu(u)
    h4 = _dot(u, w2_ref[...], (((1,), (0,)))) + b2_ref[...]
    y1 = h4 + xb

    v = _gelu(_dot(y1, f1_ref[...], (((1,), (0,)))) + bf1_ref[...])
    y2 = _dot(v, f2_ref[...], (((1,), (0,)))) + bf2_ref[...] + y1
    out_ref[0] = y2


def kernel(x, g_fc1_w, g_fc1_b, g_bn1_g, g_bn1_b, gc_w, gc_b, gc_bn_g, gc_bn_b,
           g_fc2_w, g_fc2_b, g_bn2_g, g_bn2_b,
           f_fc1_w, f_fc1_b, f_bn1_g, f_bn1_b, f_fc2_w, f_fc2_b, f_bn2_g, f_bn2_b):
    s = 1.0 / jnp.sqrt(jnp.float32(1.0 + EPS))

    def fold(w, b, g, be):
        sc = s * g
        return (w * sc[:, None]).T, (b * sc + be)[None, :]

    gcw, bgc = fold(gc_w, gc_b, gc_bn_g, gc_bn_b)
    wa = gcw[:C] - gcw[C:]
    wm = gcw[C:]
    w2, b2 = fold(g_fc2_w, g_fc2_b, g_bn2_g, g_bn2_b)
    f1, bf1 = fold(f_fc1_w, f_fc1_b, f_bn1_g, f_bn1_b)
    f2, bf2 = fold(f_fc2_w, f_fc2_b, f_bn2_g, f_bn2_b)

    xt = x.reshape(B, C, N).transpose(0, 2, 1)

    def fixed(shape):
        return pl.BlockSpec(shape, lambda b: (0,) * len(shape))

    ht_pad, idx = pl.pallas_call(
        _stage1_body,
        grid=(B,),
        in_specs=[
            pl.BlockSpec((1, N, C), lambda b: (b, 0, 0)),
            fixed((C, C)), fixed((1, C)), fixed((1, C)), fixed((1, C)),
        ],
        out_specs=[
            pl.BlockSpec((1, N, CP), lambda b: (b, 0, 0)),
            pl.BlockSpec((1, N, K), lambda b: (b, 0, 0)),
        ],
        out_shape=[
            jax.ShapeDtypeStruct((B, N, CP), _F32),
            jax.ShapeDtypeStruct((B, N, K), jnp.int32),
        ],
    )(xt, g_fc1_w.T, g_fc1_b[None, :], g_bn1_g[None, :], g_bn1_b[None, :])

    relmax = _sc_gather(ht_pad.reshape(R, CP), idx.reshape(R * K))

    out = pl.pallas_call(
        _stage3_body,
        grid=(B,),
        in_specs=[
            pl.BlockSpec((1, N, C), lambda b: (b, 0, 0)),
            pl.BlockSpec((1, N, CP), lambda b: (b, 0, 0)),
            pl.BlockSpec((1, N, CP), lambda b: (b, 0, 0)),
            fixed((C, HID)), fixed((C, HID)), fixed((1, HID)),
            fixed((HID, C)), fixed((1, C)),
            fixed((C, FFN_HID)), fixed((1, FFN_HID)),
            fixed((FFN_HID, C)), fixed((1, C)),
        ],
        out_specs=pl.BlockSpec((1, N, C), lambda b: (b, 0, 0)),
        out_shape=jax.ShapeDtypeStruct((B, N, C), _F32),
    )(xt, ht_pad, relmax.reshape(B, N, CP), wa, wm, bgc, w2, b2, f1, bf1, f2, bf2)

    return out.transpose(0, 2, 1).reshape(B, C, H, W)
